# Initial kernel scaffold; baseline (speedup 1.0000x reference)
import jax
import jax.numpy as jnp
from jax.experimental import pallas as pl


def kernel(x, edge_index, edge_weight, W1, b1, W2, b2, Wf1, bf1, Wf2, bf2):
    raise NotImplementedError("write your pallas kernel here")


# speedup vs baseline: 6.2984x; 6.2984x over previous
import functools
import jax
import jax.numpy as jnp
from jax import lax
from jax.experimental import pallas as pl
from jax.experimental.pallas import tpu as pltpu
from jax.experimental.pallas import tpu_sc as plsc

N_NODES = 10000
N_EDGES = 320000
HDIM = 32

NC = 2
NS = 16
LANES = 16

GROUPS = 4
GSZ = 4
ESUBS = NS // GROUPS
E_PER_TILE = N_EDGES // ESUBS
CHUNK = 2000
NCHUNKS = E_PER_TILE // CHUNK


def _spmm_body(yt, src_h, dst_h, w_h, out, cols, acc, sbuf, dbuf, wbuf):
    c = lax.axis_index("c")
    s = lax.axis_index("s")
    group = s // ESUBS
    esub = s % ESUBS
    fbase = c * (GROUPS * GSZ) + group * GSZ

    pltpu.sync_copy(yt.at[pl.ds(fbase * N_NODES, GSZ * N_NODES)], cols)

    zero16 = jnp.zeros((LANES,), jnp.float32)

    def zb(i, _):
        acc[pl.ds(i * LANES, LANES)] = zero16
        return 0
    lax.fori_loop(0, GSZ * N_NODES // LANES, zb, 0)

    ebase = esub * E_PER_TILE

    def chunk_body(k, _):
        off = ebase + k * CHUNK
        pltpu.sync_copy(src_h.at[pl.ds(off, CHUNK)], sbuf)
        pltpu.sync_copy(dst_h.at[pl.ds(off, CHUNK)], dbuf)
        pltpu.sync_copy(w_h.at[pl.ds(off, CHUNK)], wbuf)

        def ib(i, _):
            s16 = sbuf[pl.ds(i * LANES, LANES)]
            d16 = dbuf[pl.ds(i * LANES, LANES)]
            w16 = wbuf[pl.ds(i * LANES, LANES)]
            for j in range(GSZ):
                v = plsc.load_gather(cols, [s16 + (j * N_NODES)])
                plsc.addupdate_scatter(acc, [d16 + (j * N_NODES)], v * w16)
            return 0

        lax.fori_loop(0, CHUNK // LANES, ib, 0)
        return 0

    lax.fori_loop(0, NCHUNKS, chunk_body, 0)

    obase = esub * (HDIM * N_NODES) + fbase * N_NODES
    pltpu.sync_copy(acc, out.at[pl.ds(obase, GSZ * N_NODES)])


@jax.jit
def _spmm_partials(yt, src, dst, w):
    mesh = plsc.VectorSubcoreMesh(core_axis_name="c", subcore_axis_name="s")
    f = pl.kernel(
        _spmm_body,
        out_type=jax.ShapeDtypeStruct((ESUBS * HDIM * N_NODES,), jnp.float32),
        mesh=mesh,
        scratch_types=[
            pltpu.VMEM((GSZ * N_NODES,), jnp.float32),
            pltpu.VMEM((GSZ * N_NODES,), jnp.float32),
            pltpu.VMEM((CHUNK,), jnp.int32),
            pltpu.VMEM((CHUNK,), jnp.int32),
            pltpu.VMEM((CHUNK,), jnp.float32),
        ],
        compiler_params=pltpu.CompilerParams(needs_layout_passes=False),
    )
    return f(yt.reshape(-1), src, dst, w).reshape(ESUBS, HDIM, N_NODES)


def _spmm_t(yt, src, dst, w):
    return _spmm_partials(yt, src, dst, w).sum(axis=0)


def kernel(x, edge_index, edge_weight, W1, b1, W2, b2, Wf1, bf1, Wf2, bf2):
    src = edge_index[0]
    dst = edge_index[1]

    Yt = jnp.einsum("kfh,nf->khn", W1, x)
    Z = _spmm_t(Yt[2], src, dst, edge_weight)
    S = _spmm_t(Yt[1] + 2.0 * Z, src, dst, edge_weight)
    ht = jax.nn.relu(Yt[0] - Yt[2] + S + b1[:, None])

    Ut = jnp.einsum("kfh,fn->khn", W2, ht)
    Z2 = _spmm_t(Ut[2], src, dst, edge_weight)
    S2 = _spmm_t(Ut[1] + 2.0 * Z2, src, dst, edge_weight)
    h2t = jax.nn.relu(Ut[0] - Ut[2] + S2 + b2[:, None])

    pooled = jnp.sum(h2t, axis=1)[None, :]
    z = jax.nn.relu(pooled @ Wf1 + bf1)
    return z @ Wf2 + bf2


# speedup vs baseline: 19.2374x; 3.0544x over previous
import jax
import jax.numpy as jnp
from jax import lax
from jax.experimental import pallas as pl
from jax.experimental.pallas import tpu as pltpu
from jax.experimental.pallas import tpu_sc as plsc

N_NODES = 10000
N_EDGES = 320000
HDIM = 32

NC = 2
NS = 16
LANES = 16

GROUPS = 4
GSZ = 4
ESUBS = NS // GROUPS
E_PER_TILE = N_EDGES // ESUBS
CHUNK = 4000
NCHUNKS = E_PER_TILE // CHUNK
TOTAL_CHUNKS = N_EDGES // CHUNK
CROW = 3 * CHUNK


def _spmm_body(yt, ed, out, cols, acc, ebuf, sem0, sem1):
    c = lax.axis_index("c")
    s = lax.axis_index("s")
    group = s // ESUBS
    esub = s % ESUBS
    fbase = c * (GROUPS * GSZ) + group * GSZ
    sems = (sem0, sem1)

    cbase = esub * NCHUNKS

    pltpu.async_copy(ed.at[pl.ds(cbase * CROW, CROW)],
                     ebuf.at[pl.ds(0, CROW)], sem0)

    pltpu.sync_copy(yt.at[pl.ds(fbase * N_NODES, GSZ * N_NODES)], cols)

    zero16 = jnp.zeros((LANES,), jnp.float32)

    @plsc.parallel_loop(0, GSZ * N_NODES // LANES, unroll=8)
    def _(i):
        acc[pl.ds(i * LANES, LANES)] = zero16

    def outer(p, _):
        for b in range(2):
            k = p * 2 + b
            boff = b * CROW
            pltpu.make_async_copy(ed.at[pl.ds((cbase + k) * CROW, CROW)],
                                  ebuf.at[pl.ds(boff, CROW)], sems[b]).wait()

            @pl.when(k + 1 < NCHUNKS)
            def _():
                nb = 1 - b
                pltpu.async_copy(
                    ed.at[pl.ds((cbase + k + 1) * CROW, CROW)],
                    ebuf.at[pl.ds(nb * CROW, CROW)], sems[nb])

            @plsc.parallel_loop(0, CHUNK // LANES, unroll=4)
            def _(i):
                off = boff + i * LANES
                s16 = ebuf[pl.ds(off, LANES)]
                d16 = ebuf[pl.ds(off + CHUNK, LANES)]
                w16 = plsc.bitcast(ebuf[pl.ds(off + 2 * CHUNK, LANES)],
                                   jnp.float32)
                for j in range(GSZ):
                    v = plsc.load_gather(cols, [s16 + (j * N_NODES)])
                    plsc.addupdate_scatter(acc, [d16 + (j * N_NODES)],
                                           v * w16)
        return 0

    lax.fori_loop(0, NCHUNKS // 2, outer, 0)

    obase = esub * (HDIM * N_NODES) + fbase * N_NODES
    pltpu.sync_copy(acc, out.at[pl.ds(obase, GSZ * N_NODES)])


@jax.jit
def _spmm_partials(yt, ed):
    mesh = plsc.VectorSubcoreMesh(core_axis_name="c", subcore_axis_name="s")
    f = pl.kernel(
        _spmm_body,
        out_type=jax.ShapeDtypeStruct((ESUBS * HDIM * N_NODES,), jnp.float32),
        mesh=mesh,
        scratch_types=[
            pltpu.VMEM((GSZ * N_NODES,), jnp.float32),
            pltpu.VMEM((GSZ * N_NODES,), jnp.float32),
            pltpu.VMEM((2 * CROW,), jnp.int32),
            pltpu.SemaphoreType.DMA,
            pltpu.SemaphoreType.DMA,
        ],
        compiler_params=pltpu.CompilerParams(needs_layout_passes=False),
    )
    return f(yt, ed).reshape(ESUBS, HDIM, N_NODES)


def _spmm_t(yt, ed):
    return _spmm_partials(yt.reshape(-1), ed).sum(axis=0)


def _pack_edges(src, dst, w):
    wb = lax.bitcast_convert_type(w, jnp.int32)
    ed = jnp.stack([src.reshape(TOTAL_CHUNKS, CHUNK),
                    dst.reshape(TOTAL_CHUNKS, CHUNK),
                    wb.reshape(TOTAL_CHUNKS, CHUNK)], axis=1)
    return ed.reshape(-1)


def kernel(x, edge_index, edge_weight, W1, b1, W2, b2, Wf1, bf1, Wf2, bf2):
    src = edge_index[0]
    dst = edge_index[1]
    ed = _pack_edges(src, dst, edge_weight)

    Yt = jnp.einsum("kfh,nf->khn", W1, x)
    Z = _spmm_t(Yt[2], ed)
    S = _spmm_t(Yt[1] + 2.0 * Z, ed)
    ht = jax.nn.relu(Yt[0] - Yt[2] + S + b1[:, None])

    Ut = jnp.einsum("kfh,fn->khn", W2, ht)
    Z2 = _spmm_t(Ut[2], ed)
    S2 = _spmm_t(Ut[1] + 2.0 * Z2, ed)
    h2t = jax.nn.relu(Ut[0] - Ut[2] + S2 + b2[:, None])

    pooled = jnp.sum(h2t, axis=1)[None, :]
    z = jax.nn.relu(pooled @ Wf1 + bf1)
    return z @ Wf2 + bf2
